# Initial kernel scaffold; baseline (speedup 1.0000x reference)
import jax
import jax.numpy as jnp
from jax.experimental import pallas as pl


def kernel(node_j, node_k, phi, prob_tensor):
    raise NotImplementedError("write your pallas kernel here")


# speedup vs baseline: 3.4365x; 3.4365x over previous
import functools

import jax
import jax.numpy as jnp
from jax import lax
from jax.experimental import pallas as pl
from jax.experimental.pallas import tpu as pltpu
from jax.experimental.pallas import tpu_sc as plsc


def kernel(node_j, node_k, phi, prob_tensor):
    num_nodes, embed = phi.shape
    nchunk = embed // 16
    fdtype = phi.dtype
    mesh = plsc.VectorSubcoreMesh(core_axis_name="c", subcore_axis_name="s")

    @functools.partial(
        pl.kernel,
        out_type=jax.ShapeDtypeStruct((1,), fdtype),
        mesh=mesh,
        compiler_params=pltpu.CompilerParams(needs_layout_passes=False),
        scratch_types=[
            pltpu.VMEM((1,), jnp.int32),
            pltpu.VMEM((16,), jnp.int32),
            pltpu.VMEM((16,), jnp.int32),
            pltpu.VMEM((1, embed), fdtype),
            pltpu.VMEM((16, embed), fdtype),
            pltpu.VMEM((256,), fdtype),
            pltpu.VMEM((16,), fdtype),
            pltpu.SemaphoreType.DMA,
        ],
    )
    def run(node_j_hbm, node_k_hbm, phi_hbm, prob_hbm, out_hbm,
            jv_v, kv_v, idx_v, h_v, rows_v, part_v, f_v, sem):
        cid = lax.axis_index("c")
        sid = lax.axis_index("s")

        @pl.when(jnp.logical_and(cid == 0, sid == 0))
        def _():
            pltpu.sync_copy(node_j_hbm, jv_v)
            pltpu.sync_copy(node_k_hbm, kv_v.at[pl.ds(0, 1)])
            h_cp = pltpu.async_copy(phi_hbm.at[jv_v], h_v, sem)

            w = kv_v[...][0] + num_nodes
            w_vec = jnp.broadcast_to(w, (16,))
            iota = lax.iota(jnp.int32, 16)
            s_vec = iota + 1
            idx_vec = lax.shift_right_logical(w_vec, s_vec)
            bits = lax.shift_right_logical(w_vec, iota) & 1
            sign_f = (1 - 2 * bits).astype(fdtype)
            active = idx_vec >= 2

            idx_v[...] = idx_vec
            rows_cp = pltpu.async_copy(prob_hbm.at[idx_v], rows_v, sem)
            h_cp.wait()
            rows_cp.wait()

            hc = [h_v[0, pl.ds(16 * c, 16)] for c in range(nchunk)]
            for r in range(16):
                acc = rows_v[r, pl.ds(0, 16)] * hc[0]
                for c in range(1, nchunk):
                    acc = acc + rows_v[r, pl.ds(16 * c, 16)] * hc[c]
                part_v[pl.ds(16 * r, 16)] = acc

            row_base = iota * 16
            dots = plsc.load_gather(part_v, [row_base])
            for k in range(1, 16):
                dots = dots + plsc.load_gather(part_v, [row_base + k])

            x = sign_f * dots
            f = 1.0 / (1.0 + jnp.exp(-x))
            f = jnp.where(active, f, jnp.ones((16,), fdtype))

            f_v[...] = f
            v = f
            for step in (8, 4, 2, 1):
                v = v * plsc.load_gather(f_v, [iota ^ step])
                f_v[...] = v
            pltpu.sync_copy(f_v.at[pl.ds(0, 1)], out_hbm)

    return run(node_j, node_k, phi, prob_tensor)


# speedup vs baseline: 3.6579x; 1.0644x over previous
import functools

import jax
import jax.numpy as jnp
from jax import lax
from jax.experimental import pallas as pl
from jax.experimental.pallas import tpu as pltpu
from jax.experimental.pallas import tpu_sc as plsc


def kernel(node_j, node_k, phi, prob_tensor):
    num_nodes, embed = phi.shape
    nchunk = embed // 16
    fdtype = phi.dtype
    mesh = plsc.VectorSubcoreMesh(
        core_axis_name="c", subcore_axis_name="s", num_cores=1)

    @functools.partial(
        pl.kernel,
        out_type=jax.ShapeDtypeStruct((1,), fdtype),
        mesh=mesh,
        compiler_params=pltpu.CompilerParams(needs_layout_passes=False),
        scratch_types=[
            pltpu.VMEM((1,), jnp.int32),
            pltpu.VMEM((16,), jnp.int32),
            pltpu.VMEM((16,), jnp.int32),
            pltpu.VMEM((1, embed), fdtype),
            pltpu.VMEM((16, embed), fdtype),
            pltpu.VMEM((256,), fdtype),
            pltpu.VMEM((16,), fdtype),
            pltpu.SemaphoreType.DMA,
        ],
    )
    def run(node_j_hbm, node_k_hbm, phi_hbm, prob_hbm, out_hbm,
            jv_v, kv_v, idx_v, h_v, rows_v, part_v, f_v, sem):
        cid = lax.axis_index("c")
        sid = lax.axis_index("s")

        @pl.when(jnp.logical_and(cid == 0, sid == 0))
        def _():
            pltpu.sync_copy(node_j_hbm, jv_v)
            pltpu.sync_copy(node_k_hbm, kv_v.at[pl.ds(0, 1)])
            h_cp = pltpu.async_copy(phi_hbm.at[jv_v], h_v, sem)

            w = kv_v[...][0] + num_nodes
            w_vec = jnp.broadcast_to(w, (16,))
            iota = lax.iota(jnp.int32, 16)
            s_vec = iota + 1
            idx_vec = lax.shift_right_logical(w_vec, s_vec)
            bits = lax.shift_right_logical(w_vec, iota) & 1
            sign_f = (1 - 2 * bits).astype(fdtype)
            active = idx_vec >= 2

            idx_v[...] = idx_vec
            rows_cp = pltpu.async_copy(prob_hbm.at[idx_v], rows_v, sem)
            h_cp.wait()
            rows_cp.wait()

            hc = [h_v[0, pl.ds(16 * c, 16)] for c in range(nchunk)]
            for r in range(16):
                acc = rows_v[r, pl.ds(0, 16)] * hc[0]
                for c in range(1, nchunk):
                    acc = acc + rows_v[r, pl.ds(16 * c, 16)] * hc[c]
                part_v[pl.ds(16 * r, 16)] = acc

            row_base = iota * 16
            dots = plsc.load_gather(part_v, [row_base])
            for k in range(1, 16):
                dots = dots + plsc.load_gather(part_v, [row_base + k])

            x = sign_f * dots
            f = 1.0 / (1.0 + jnp.exp(-x))
            f = jnp.where(active, f, jnp.ones((16,), fdtype))

            f_v[...] = f
            v = f
            for step in (8, 4, 2, 1):
                v = v * plsc.load_gather(f_v, [iota ^ step])
                f_v[...] = v
            pltpu.sync_copy(f_v.at[pl.ds(0, 1)], out_hbm)

    return run(node_j, node_k, phi, prob_tensor)


# speedup vs baseline: 3.7970x; 1.0380x over previous
import functools

import jax
import jax.numpy as jnp
from jax import lax
from jax.experimental import pallas as pl
from jax.experimental.pallas import tpu as pltpu
from jax.experimental.pallas import tpu_sc as plsc


def kernel(node_j, node_k, phi, prob_tensor):
    num_nodes, embed = phi.shape
    nchunk = embed // 16
    fdtype = phi.dtype
    mesh = plsc.VectorSubcoreMesh(
        core_axis_name="c", subcore_axis_name="s", num_cores=1,
        num_subcores=1)

    @functools.partial(
        pl.kernel,
        out_type=jax.ShapeDtypeStruct((1,), fdtype),
        mesh=mesh,
        compiler_params=pltpu.CompilerParams(needs_layout_passes=False),
        scratch_types=[
            pltpu.VMEM((1,), jnp.int32),
            pltpu.VMEM((16,), jnp.int32),
            pltpu.VMEM((16,), jnp.int32),
            pltpu.VMEM((1, embed), fdtype),
            pltpu.VMEM((16, embed), fdtype),
            pltpu.VMEM((256,), fdtype),
            pltpu.VMEM((16,), fdtype),
            pltpu.SemaphoreType.DMA,
            pltpu.SemaphoreType.DMA,
        ],
    )
    def run(node_j_hbm, node_k_hbm, phi_hbm, prob_hbm, out_hbm,
            jv_v, kv_v, idx_v, h_v, rows_v, part_v, f_v, sem, sem2):
        cid = lax.axis_index("c")
        sid = lax.axis_index("s")

        @pl.when(jnp.logical_and(cid == 0, sid == 0))
        def _():
            j_cp = pltpu.async_copy(node_j_hbm, jv_v, sem)
            k_cp = pltpu.async_copy(node_k_hbm, kv_v.at[pl.ds(0, 1)], sem2)
            k_cp.wait()

            w = kv_v[...][0] + num_nodes
            w_vec = jnp.broadcast_to(w, (16,))
            iota = lax.iota(jnp.int32, 16)
            idx_vec = lax.shift_right_logical(w_vec, iota + 1)
            idx_v[...] = idx_vec
            rows_cp = pltpu.async_copy(prob_hbm.at[idx_v], rows_v, sem2)
            j_cp.wait()
            h_cp = pltpu.async_copy(phi_hbm.at[jv_v], h_v, sem)

            bits = lax.shift_right_logical(w_vec, iota) & 1
            sign_f = (1 - 2 * bits).astype(fdtype)
            active = idx_vec >= 2

            h_cp.wait()
            rows_cp.wait()

            hc = [h_v[0, pl.ds(16 * c, 16)] for c in range(nchunk)]
            for r in range(16):
                acc = rows_v[r, pl.ds(0, 16)] * hc[0]
                for c in range(1, nchunk):
                    acc = acc + rows_v[r, pl.ds(16 * c, 16)] * hc[c]
                part_v[pl.ds(16 * r, 16)] = acc

            row_base = iota * 16
            dots = plsc.load_gather(part_v, [row_base])
            for k in range(1, 16):
                dots = dots + plsc.load_gather(part_v, [row_base + k])

            x = sign_f * dots
            f = 1.0 / (1.0 + jnp.exp(-x))
            f = jnp.where(active, f, jnp.ones((16,), fdtype))

            f_v[...] = f
            v = f
            for step in (8, 4, 2, 1):
                v = v * plsc.load_gather(f_v, [iota ^ step])
                f_v[...] = v
            pltpu.sync_copy(f_v.at[pl.ds(0, 1)], out_hbm)

    return run(node_j, node_k, phi, prob_tensor)
